# baseline (speedup 1.0000x reference)
import jax
import jax.numpy as jnp
import numpy as np
from jax.experimental import pallas as pl

_R = 8
_I32_MIN = np.int32(-2147483648)
_I32_MAX = np.int32(2147483647)
_INCS = [np.int32(np.uint32(1 << b).astype(np.int32)) for b in range(31, -1, -1)]


def _sampler_kernel(temp_ref, topp_ref, topk_ref, minp_ref, logits_ref, q_ref,
                    lp_ref, gidx_ref, ridx_ref, *, n_greedy_blocks):
    i = pl.program_id(0)
    x = logits_ref[...] / temp_ref[...]
    m = jnp.max(x, axis=1, keepdims=True)
    w = jnp.exp(x - m)
    bits = jax.lax.bitcast_convert_type(x, jnp.int32)
    c = jnp.where(bits < 0, bits ^ np.int32(0x7FFFFFFF), bits)

    ones = jnp.ones((x.shape[1], 1), jnp.float32)

    kf = topk_ref[...].astype(jnp.float32)
    t = jnp.full(kf.shape, _I32_MIN, jnp.int32)
    for inc in _INCS:
        cand = t + inc
        cnt = jnp.dot(jnp.where(c >= cand, 1.0, 0.0), ones,
                      preferred_element_type=jnp.float32)
        t = jnp.where((cand > t) & (cnt >= kf), cand, t)
    keep_k = c >= t
    w_k = jnp.where(keep_k, w, 0.0)
    zk = jnp.dot(w_k, ones, precision=jax.lax.Precision.HIGHEST,
                 preferred_element_type=jnp.float32)
    pz = topp_ref[...] * zk

    t2 = t - 1
    for inc in _INCS:
        cand = t2 + inc
        s = jnp.dot(jnp.where(c > cand, w_k, 0.0), ones,
                    precision=jax.lax.Precision.HIGHEST,
                    preferred_element_type=jnp.float32)
        t2 = jnp.where((cand > t2) & (s >= pz), cand, t2)

    kept = keep_k & (c > t2) & (w >= minp_ref[...])
    w_f = jnp.where(kept, w, 0.0)
    zf = jnp.sum(w_f, axis=1, keepdims=True)
    lp_ref[...] = jnp.where(kept, x - m - jnp.log(zf), -jnp.inf)

    iota = jax.lax.broadcasted_iota(jnp.int32, x.shape, 1)

    @pl.when(i < n_greedy_blocks)
    def _greedy():
        gidx_ref[...] = jnp.min(jnp.where(x == m, iota, _I32_MAX),
                                axis=1, keepdims=True)

    @pl.when(i >= n_greedy_blocks)
    def _random():
        rat = w_f / q_ref[...]
        nan_mask = jnp.isnan(rat)
        nan_idx = jnp.min(jnp.where(nan_mask, iota, _I32_MAX),
                          axis=1, keepdims=True)
        rat_c = jnp.where(nan_mask, -jnp.inf, rat)
        rmax = jnp.max(rat_c, axis=1, keepdims=True)
        ridx = jnp.min(jnp.where(rat_c == rmax, iota, _I32_MAX),
                       axis=1, keepdims=True)
        ridx_ref[...] = jnp.where(nan_idx < _I32_MAX, nan_idx, ridx)


def kernel(logits, temperatures, top_ps, top_ks, min_ps,
           greedy_indices, random_indices, q):
    B, V = logits.shape
    NQ = q.shape[0]
    n_greedy_blocks = (B - NQ) // _R
    import functools
    body = functools.partial(_sampler_kernel, n_greedy_blocks=n_greedy_blocks)
    row_spec = pl.BlockSpec((_R, 1), lambda i: (i, 0))
    big_spec = pl.BlockSpec((_R, V), lambda i: (i, 0))
    q_spec = pl.BlockSpec((_R, V),
                          lambda i: (jnp.maximum(i - n_greedy_blocks, 0), 0))
    lp, gidx, ridx = pl.pallas_call(
        body,
        grid=(B // _R,),
        in_specs=[row_spec, row_spec, row_spec, row_spec, big_spec, q_spec],
        out_specs=[big_spec,
                   pl.BlockSpec((_R, 1), lambda i: (i, 0)),
                   pl.BlockSpec((_R, 1), lambda i: (i, 0))],
        out_shape=[jax.ShapeDtypeStruct((B, V), jnp.float32),
                   jax.ShapeDtypeStruct((B, 1), jnp.int32),
                   jax.ShapeDtypeStruct((B, 1), jnp.int32)],
    )(temperatures.astype(jnp.float32).reshape(B, 1),
      top_ps.astype(jnp.float32).reshape(B, 1),
      top_ks.astype(jnp.int32).reshape(B, 1),
      min_ps.astype(jnp.float32).reshape(B, 1),
      logits.astype(jnp.float32), q)
    greedy = jnp.take(gidx[:, 0], greedy_indices)
    multinomial = jnp.take(ridx[:, 0], random_indices).reshape(-1, 1)
    return (lp, greedy, multinomial)


# speedup vs baseline: 7.4145x; 7.4145x over previous
import jax
import jax.numpy as jnp
import numpy as np
from jax.experimental import pallas as pl

_R = 8
_I32_MIN = np.int32(-2147483648)
_I32_MAX = np.int32(2147483647)
_INCS = [np.int32(np.uint32(1 << b).astype(np.int32)) for b in range(31, -1, -1)]


def _sampler_kernel(temp_ref, topp_ref, topk_ref, minp_ref, logits_ref, q_ref,
                    lp_ref, gidx_ref, ridx_ref, *, n_greedy_blocks):
    i = pl.program_id(0)
    x = logits_ref[...] / temp_ref[...]
    m = jnp.max(x, axis=1, keepdims=True)
    bits = jax.lax.bitcast_convert_type(x, jnp.int32)
    c = jnp.where(bits < 0, bits ^ np.int32(0x7FFFFFFF), bits)

    kf = topk_ref[...].astype(jnp.float32)
    t = jnp.full(kf.shape, _I32_MIN, jnp.int32)
    for inc in _INCS:
        cand = t + inc
        cnt = jnp.sum(jnp.where(c >= cand, 1.0, 0.0), axis=1, keepdims=True)
        t = jnp.where((cand > t) & (cnt >= kf), cand, t)
    keep_k = c >= t
    w = jnp.exp(x - m)
    w_k = jnp.where(keep_k, w, 0.0)
    wb = jax.lax.bitcast_convert_type(w_k, jnp.int32)
    zk = jnp.sum(w_k, axis=1, keepdims=True)
    pz = topp_ref[...] * zk

    t2 = jnp.zeros(kf.shape, jnp.int32)
    for b in range(29, -1, -1):
        cand = t2 + np.int32(1 << b)
        s = jnp.sum(jnp.where(wb > cand, w_k, 0.0), axis=1, keepdims=True)
        t2 = jnp.where(s >= pz, cand, t2)

    kept = keep_k & (wb > t2) & (w >= minp_ref[...])
    w_f = jnp.where(kept, w, 0.0)
    zf = jnp.sum(w_f, axis=1, keepdims=True)
    lp_ref[...] = jnp.where(kept, x - m - jnp.log(zf), -jnp.inf)

    iota = jax.lax.broadcasted_iota(jnp.int32, x.shape, 1)

    @pl.when(i < n_greedy_blocks)
    def _greedy():
        gidx_ref[...] = jnp.min(jnp.where(x == m, iota, _I32_MAX),
                                axis=1, keepdims=True)

    @pl.when(i >= n_greedy_blocks)
    def _random():
        rat = w_f / q_ref[...]
        nan_mask = jnp.isnan(rat)
        nan_idx = jnp.min(jnp.where(nan_mask, iota, _I32_MAX),
                          axis=1, keepdims=True)
        rat_c = jnp.where(nan_mask, -jnp.inf, rat)
        rmax = jnp.max(rat_c, axis=1, keepdims=True)
        ridx = jnp.min(jnp.where(rat_c == rmax, iota, _I32_MAX),
                       axis=1, keepdims=True)
        ridx_ref[...] = jnp.where(nan_idx < _I32_MAX, nan_idx, ridx)


def kernel(logits, temperatures, top_ps, top_ks, min_ps,
           greedy_indices, random_indices, q):
    B, V = logits.shape
    NQ = q.shape[0]
    n_greedy_blocks = (B - NQ) // _R
    import functools
    body = functools.partial(_sampler_kernel, n_greedy_blocks=n_greedy_blocks)
    row_spec = pl.BlockSpec((_R, 1), lambda i: (i, 0))
    big_spec = pl.BlockSpec((_R, V), lambda i: (i, 0))
    q_spec = pl.BlockSpec((_R, V),
                          lambda i: (jnp.maximum(i - n_greedy_blocks, 0), 0))
    lp, gidx, ridx = pl.pallas_call(
        body,
        grid=(B // _R,),
        in_specs=[row_spec, row_spec, row_spec, row_spec, big_spec, q_spec],
        out_specs=[big_spec,
                   pl.BlockSpec((_R, 1), lambda i: (i, 0)),
                   pl.BlockSpec((_R, 1), lambda i: (i, 0))],
        out_shape=[jax.ShapeDtypeStruct((B, V), jnp.float32),
                   jax.ShapeDtypeStruct((B, 1), jnp.int32),
                   jax.ShapeDtypeStruct((B, 1), jnp.int32)],
    )(temperatures.astype(jnp.float32).reshape(B, 1),
      top_ps.astype(jnp.float32).reshape(B, 1),
      top_ks.astype(jnp.int32).reshape(B, 1),
      min_ps.astype(jnp.float32).reshape(B, 1),
      logits.astype(jnp.float32), q)
    greedy = jnp.take(gidx[:, 0], greedy_indices)
    multinomial = jnp.take(ridx[:, 0], random_indices).reshape(-1, 1)
    return (lp, greedy, multinomial)


# speedup vs baseline: 13.9797x; 1.8855x over previous
import jax
import jax.numpy as jnp
import numpy as np
from jax.experimental import pallas as pl

_R = 8
_I32_MIN = np.int32(-2147483648)
_I32_MAX = np.int32(2147483647)
_INCS = [np.int32(np.uint32(1 << b).astype(np.int32)) for b in range(31, -1, -1)]


def _rsum(v):
    n = v.shape[1]
    cut = (n // (8 * 128)) * 128
    parts = []
    s0 = 0
    for j in range(7):
        parts.append(jnp.sum(jax.lax.slice_in_dim(v, s0, s0 + cut, axis=1),
                             axis=1, keepdims=True))
        s0 += cut
    parts.append(jnp.sum(jax.lax.slice_in_dim(v, s0, n, axis=1),
                         axis=1, keepdims=True))
    return (((parts[0] + parts[1]) + (parts[2] + parts[3]))
            + ((parts[4] + parts[5]) + (parts[6] + parts[7])))


def _sampler_kernel(temp_ref, topp_ref, topk_ref, minp_ref, logits_ref, q_ref,
                    lp_ref, gidx_ref, ridx_ref, *, n_greedy_blocks):
    i = pl.program_id(0)
    x = logits_ref[...] / temp_ref[...]
    m = jnp.max(x, axis=1, keepdims=True)
    bits = jax.lax.bitcast_convert_type(x, jnp.int32)
    c = jnp.where(bits < 0, bits ^ np.int32(0x7FFFFFFF), bits)

    kf = topk_ref[...].astype(jnp.float32)
    t = jnp.full(kf.shape, _I32_MIN, jnp.int32)
    for inc in _INCS:
        cand = t + inc
        cnt = _rsum(jnp.where(c >= cand, 1.0, 0.0))
        t = jnp.where((cand > t) & (cnt >= kf), cand, t)
    keep_k = c >= t
    w = jnp.exp(x - m)
    w_k = jnp.where(keep_k, w, 0.0)
    wb = jax.lax.bitcast_convert_type(w_k, jnp.int32)
    zk = _rsum(w_k)
    pz = topp_ref[...] * zk

    t2 = jnp.zeros(kf.shape, jnp.int32)
    for b in range(29, -1, -1):
        cand = t2 + np.int32(1 << b)
        s = _rsum(jnp.where(wb > cand, w_k, 0.0))
        t2 = jnp.where(s >= pz, cand, t2)

    kept = keep_k & (wb > t2) & (w >= minp_ref[...])
    w_f = jnp.where(kept, w, 0.0)
    zf = jnp.sum(w_f, axis=1, keepdims=True)
    lp_ref[...] = jnp.where(kept, x - m - jnp.log(zf), -jnp.inf)

    iota = jax.lax.broadcasted_iota(jnp.int32, x.shape, 1)

    @pl.when(i < n_greedy_blocks)
    def _greedy():
        gidx_ref[...] = jnp.min(jnp.where(x == m, iota, _I32_MAX),
                                axis=1, keepdims=True)

    @pl.when(i >= n_greedy_blocks)
    def _random():
        rat = w_f / q_ref[...]
        nan_mask = jnp.isnan(rat)
        nan_idx = jnp.min(jnp.where(nan_mask, iota, _I32_MAX),
                          axis=1, keepdims=True)
        rat_c = jnp.where(nan_mask, -jnp.inf, rat)
        rmax = jnp.max(rat_c, axis=1, keepdims=True)
        ridx = jnp.min(jnp.where(rat_c == rmax, iota, _I32_MAX),
                       axis=1, keepdims=True)
        ridx_ref[...] = jnp.where(nan_idx < _I32_MAX, nan_idx, ridx)


def kernel(logits, temperatures, top_ps, top_ks, min_ps,
           greedy_indices, random_indices, q):
    B, V = logits.shape
    NQ = q.shape[0]
    n_greedy_blocks = (B - NQ) // _R
    import functools
    body = functools.partial(_sampler_kernel, n_greedy_blocks=n_greedy_blocks)
    row_spec = pl.BlockSpec((_R, 1), lambda i: (i, 0))
    big_spec = pl.BlockSpec((_R, V), lambda i: (i, 0))
    q_spec = pl.BlockSpec((_R, V),
                          lambda i: (jnp.maximum(i - n_greedy_blocks, 0), 0))
    lp, gidx, ridx = pl.pallas_call(
        body,
        grid=(B // _R,),
        in_specs=[row_spec, row_spec, row_spec, row_spec, big_spec, q_spec],
        out_specs=[big_spec,
                   pl.BlockSpec((_R, 1), lambda i: (i, 0)),
                   pl.BlockSpec((_R, 1), lambda i: (i, 0))],
        out_shape=[jax.ShapeDtypeStruct((B, V), jnp.float32),
                   jax.ShapeDtypeStruct((B, 1), jnp.int32),
                   jax.ShapeDtypeStruct((B, 1), jnp.int32)],
    )(temperatures.astype(jnp.float32).reshape(B, 1),
      top_ps.astype(jnp.float32).reshape(B, 1),
      top_ks.astype(jnp.int32).reshape(B, 1),
      min_ps.astype(jnp.float32).reshape(B, 1),
      logits.astype(jnp.float32), q)
    greedy = jnp.take(gidx[:, 0], greedy_indices)
    multinomial = jnp.take(ridx[:, 0], random_indices).reshape(-1, 1)
    return (lp, greedy, multinomial)


# speedup vs baseline: 14.7979x; 1.0585x over previous
import jax
import jax.numpy as jnp
import numpy as np
from jax.experimental import pallas as pl

_R = 8
_I32_MIN = np.int32(-2147483648)
_I32_MAX = np.int32(2147483647)
_INCS = [np.int32(np.uint32(1 << b).astype(np.int32)) for b in range(31, -1, -1)]


def _rsum(v):
    n = v.shape[1]
    nparts = 16
    cut = (n // (nparts * 128)) * 128
    parts = []
    s0 = 0
    for j in range(nparts - 1):
        parts.append(jnp.sum(jax.lax.slice_in_dim(v, s0, s0 + cut, axis=1),
                             axis=1, keepdims=True))
        s0 += cut
    parts.append(jnp.sum(jax.lax.slice_in_dim(v, s0, n, axis=1),
                         axis=1, keepdims=True))
    while len(parts) > 1:
        parts = [parts[j] + parts[j + 1] for j in range(0, len(parts), 2)]
    return parts[0]


def _sampler_kernel(temp_ref, topp_ref, topk_ref, minp_ref, logits_ref, q_ref,
                    lp_ref, gidx_ref, ridx_ref, *, n_greedy_blocks):
    i = pl.program_id(0)
    x = logits_ref[...] / temp_ref[...]
    m = jnp.max(x, axis=1, keepdims=True)
    bits = jax.lax.bitcast_convert_type(x, jnp.int32)
    c = jnp.where(bits < 0, bits ^ np.int32(0x7FFFFFFF), bits)

    kf = topk_ref[...].astype(jnp.float32)
    t = jnp.full(kf.shape, _I32_MIN, jnp.int32)
    for inc in _INCS:
        cand = t + inc
        cnt = _rsum(jnp.where(c >= cand, 1.0, 0.0))
        t = jnp.where((cand > t) & (cnt >= kf), cand, t)
    keep_k = c >= t
    w = jnp.exp(x - m)
    w_k = jnp.where(keep_k, w, 0.0)
    wb = jax.lax.bitcast_convert_type(w_k, jnp.int32)
    zk = _rsum(w_k)
    pz = topp_ref[...] * zk

    t2 = jnp.zeros(kf.shape, jnp.int32)
    for b in range(29, -1, -1):
        cand = t2 + np.int32(1 << b)
        s = _rsum(jnp.where(wb > cand, w_k, 0.0))
        t2 = jnp.where(s >= pz, cand, t2)

    kept = keep_k & (wb > t2) & (w >= minp_ref[...])
    w_f = jnp.where(kept, w, 0.0)
    zf = jnp.sum(w_f, axis=1, keepdims=True)
    lp_ref[...] = jnp.where(kept, x - m - jnp.log(zf), -jnp.inf)

    iota = jax.lax.broadcasted_iota(jnp.int32, x.shape, 1)

    @pl.when(i < n_greedy_blocks)
    def _greedy():
        gidx_ref[...] = jnp.min(jnp.where(x == m, iota, _I32_MAX),
                                axis=1, keepdims=True)

    @pl.when(i >= n_greedy_blocks)
    def _random():
        rat = w_f / q_ref[...]
        nan_mask = jnp.isnan(rat)
        nan_idx = jnp.min(jnp.where(nan_mask, iota, _I32_MAX),
                          axis=1, keepdims=True)
        rat_c = jnp.where(nan_mask, -jnp.inf, rat)
        rmax = jnp.max(rat_c, axis=1, keepdims=True)
        ridx = jnp.min(jnp.where(rat_c == rmax, iota, _I32_MAX),
                       axis=1, keepdims=True)
        ridx_ref[...] = jnp.where(nan_idx < _I32_MAX, nan_idx, ridx)


def kernel(logits, temperatures, top_ps, top_ks, min_ps,
           greedy_indices, random_indices, q):
    B, V = logits.shape
    NQ = q.shape[0]
    n_greedy_blocks = (B - NQ) // _R
    import functools
    body = functools.partial(_sampler_kernel, n_greedy_blocks=n_greedy_blocks)
    row_spec = pl.BlockSpec((_R, 1), lambda i: (i, 0))
    big_spec = pl.BlockSpec((_R, V), lambda i: (i, 0))
    q_spec = pl.BlockSpec((_R, V),
                          lambda i: (jnp.maximum(i - n_greedy_blocks, 0), 0))
    lp, gidx, ridx = pl.pallas_call(
        body,
        grid=(B // _R,),
        in_specs=[row_spec, row_spec, row_spec, row_spec, big_spec, q_spec],
        out_specs=[big_spec,
                   pl.BlockSpec((_R, 1), lambda i: (i, 0)),
                   pl.BlockSpec((_R, 1), lambda i: (i, 0))],
        out_shape=[jax.ShapeDtypeStruct((B, V), jnp.float32),
                   jax.ShapeDtypeStruct((B, 1), jnp.int32),
                   jax.ShapeDtypeStruct((B, 1), jnp.int32)],
    )(temperatures.astype(jnp.float32).reshape(B, 1),
      top_ps.astype(jnp.float32).reshape(B, 1),
      top_ks.astype(jnp.int32).reshape(B, 1),
      min_ps.astype(jnp.float32).reshape(B, 1),
      logits.astype(jnp.float32), q)
    greedy = jnp.take(gidx[:, 0], greedy_indices)
    multinomial = jnp.take(ridx[:, 0], random_indices).reshape(-1, 1)
    return (lp, greedy, multinomial)
